# Initial kernel scaffold; baseline (speedup 1.0000x reference)
import jax
import jax.numpy as jnp
from jax.experimental import pallas as pl


def kernel(transient_params, noise):
    raise NotImplementedError("write your pallas kernel here")


# speedup vs baseline: 81.2707x; 81.2707x over previous
import functools

import jax
import jax.numpy as jnp
from jax import lax
from jax.experimental import pallas as pl
from jax.experimental.pallas import tpu as pltpu
from jax.experimental.pallas import tpu_sc as plsc

SAMPLE_RATE = 16000
BLOCK = 64
MAX_DELAY = 480
N_VOICES = 16 * 512
NUM_WORKERS = 32
VPW = N_VOICES // NUM_WORKERS
GROUPS = VPW // 16


def _sc_body(noise_hbm, p0_hbm, p1_hbm, p2_hbm, p3_hbm, out_hbm,
             noise_v, out_v, p0_v, p1_v, p2_v, p3_v):
    wid = lax.axis_index("s") * 2 + lax.axis_index("c")
    base = wid * VPW
    pltpu.sync_copy(noise_hbm.at[wid], noise_v)
    pltpu.sync_copy(p0_hbm.at[pl.ds(base, VPW)], p0_v)
    pltpu.sync_copy(p1_hbm.at[pl.ds(base, VPW)], p1_v)
    pltpu.sync_copy(p2_hbm.at[pl.ds(base, VPW)], p2_v)
    pltpu.sync_copy(p3_hbm.at[pl.ds(base, VPW)], p3_v)

    def group(g, _):
        sl = pl.ds(g * 16, 16)
        pa = p0_v[sl]
        en = p1_v[sl]
        pt = p2_v[sl]
        pb = p3_v[sl]
        attack_samples = jnp.maximum((0.0005 + pa * 0.0495) * SAMPLE_RATE, 1.0)
        tau = jnp.maximum(attack_samples, 1.0)
        rho = jnp.exp(-1.0 / tau)
        tilt = pt * 2.0 - 1.0
        bandwidth = 0.05 + pb * 0.95
        dly = jnp.clip((BLOCK * (0.5 + 0.5 * bandwidth)).astype(jnp.int32),
                       1, MAX_DELAY)
        col = g * 16 + lax.iota(jnp.int32, 16)

        def step(s, carry):
            env, acc, svec = carry
            off = s * VPW + g * 16
            nz = noise_v[pl.ds(off, 16)]
            burst = nz * env * en
            row = svec - dly
            msk = row >= 0
            gidx = jnp.where(msk, row, 0) * VPW + col
            prev = plsc.load_gather(out_v, [gidx], mask=msk)
            y = burst + tilt * jnp.where(msk, prev, 0.0)
            out_v[pl.ds(off, 16)] = y
            return (env * rho, acc + y * y, svec + 1)

        env0 = jnp.ones((16,), jnp.float32)
        acc0 = jnp.zeros((16,), jnp.float32)
        sv0 = jnp.zeros((16,), jnp.int32)
        _, acc, _ = lax.fori_loop(0, BLOCK, step, (env0, acc0, sv0))

        m = acc * (1.0 / BLOCK) + 1e-5
        bits = plsc.bitcast(m, jnp.int32)
        r = plsc.bitcast(0x5F3759DF - (bits >> 1), jnp.float32)
        for _ in range(4):
            r = r * (1.5 - 0.5 * m * r * r)

        def scale(s, c):
            off = s * VPW + g * 16
            out_v[pl.ds(off, 16)] = out_v[pl.ds(off, 16)] * r
            return c

        lax.fori_loop(0, BLOCK, scale, 0)
        return 0

    lax.fori_loop(0, GROUPS, group, 0)
    pltpu.sync_copy(out_v, out_hbm.at[wid])


_sc_call = pl.kernel(
    _sc_body,
    out_type=jax.ShapeDtypeStruct((NUM_WORKERS, BLOCK * VPW), jnp.float32),
    mesh=plsc.VectorSubcoreMesh(core_axis_name="c", subcore_axis_name="s"),
    compiler_params=pltpu.CompilerParams(needs_layout_passes=False),
    scratch_types=[
        pltpu.VMEM((BLOCK * VPW,), jnp.float32),
        pltpu.VMEM((BLOCK * VPW,), jnp.float32),
        pltpu.VMEM((VPW,), jnp.float32),
        pltpu.VMEM((VPW,), jnp.float32),
        pltpu.VMEM((VPW,), jnp.float32),
        pltpu.VMEM((VPW,), jnp.float32),
    ],
)


def kernel(transient_params, noise):
    Bb, Tt, _ = transient_params.shape
    p = transient_params.reshape(N_VOICES, 4)
    noise_t = (noise.reshape(N_VOICES, BLOCK).T
               .reshape(BLOCK, NUM_WORKERS, VPW).transpose(1, 0, 2)
               .reshape(NUM_WORKERS, BLOCK * VPW))
    out3 = _sc_call(noise_t, p[:, 0], p[:, 1], p[:, 2], p[:, 3])
    out = (out3.reshape(NUM_WORKERS, BLOCK, VPW)
           .transpose(1, 0, 2).reshape(BLOCK, N_VOICES).T)
    return out.reshape(Bb, Tt * BLOCK)
